# baseline (speedup 1.0000x reference)
import functools

import jax
import jax.numpy as jnp
from jax import lax
from jax.experimental import pallas as pl
from jax.experimental.pallas import tpu as pltpu
from jax.experimental.pallas import tpu_sc as plsc

NC = 2
NS = 16
L = 16
CH = 128
SH = 14
DMASK = (1 << SH) - 1
RING = 4

_dnums = lax.GatherDimensionNumbers(
    offset_dims=(), collapsed_slice_dims=(0,), start_index_map=(0,)
)


def _bcast(vec, i):
    return lax.gather(
        vec, jnp.full((L, 1), i, jnp.int32), _dnums, (1,),
        mode=lax.GatherScatterMode.PROMISE_IN_BOUNDS,
    )


def _linear(x, Wt, b2):
    M, Din = x.shape
    Dout = Wt.shape[1]
    BM = 1000

    def body(x_ref, wt_ref, b_ref, o_ref):
        o_ref[...] = (
            jnp.dot(x_ref[...], wt_ref[...], preferred_element_type=jnp.float32)
            + b_ref[...]
        )

    return pl.pallas_call(
        body,
        grid=(M // BM,),
        in_specs=[
            pl.BlockSpec((BM, Din), lambda i: (i, 0)),
            pl.BlockSpec((Din, Dout), lambda i: (0, 0)),
            pl.BlockSpec((1, Dout), lambda i: (0, 0)),
        ],
        out_specs=pl.BlockSpec((BM, Dout), lambda i: (i, 0)),
        out_shape=jax.ShapeDtypeStruct((M, Dout), jnp.float32),
    )(x, Wt, b2)


def _concat_relu(p0, p1, n):
    Dh = p0.shape[1]
    BM = 1000

    def body(a_ref, b_ref, o_ref):
        o_ref[:, :Dh] = jnp.maximum(a_ref[...], 0.0)
        o_ref[:, Dh:] = jnp.maximum(b_ref[...], 0.0)

    return pl.pallas_call(
        body,
        grid=(n // BM,),
        in_specs=[
            pl.BlockSpec((BM, Dh), lambda i: (i, 0)),
            pl.BlockSpec((BM, Dh), lambda i: (i, 0)),
        ],
        out_specs=pl.BlockSpec((BM, 2 * Dh), lambda i: (i, 0)),
        out_shape=jax.ShapeDtypeStruct((n, 2 * Dh), jnp.float32),
    )(p0, p1)


def _spmm_sc(hpair, sd3, wf3, np_pair):
    D = hpair.shape[2]
    HP = hpair.shape[1]
    K = sd3.shape[1]
    SEG = HP // NS
    AS = np_pair // NS
    mesh = plsc.VectorSubcoreMesh(core_axis_name="c", subcore_axis_name="s")

    @functools.partial(
        pl.kernel,
        mesh=mesh,
        out_type=jax.ShapeDtypeStruct((NC, np_pair, D), jnp.float32),
        scratch_types=[
            pltpu.VMEM((RING, CH), jnp.int32),
            pltpu.VMEM((RING, CH), jnp.float32),
            pltpu.VMEM((2, CH), jnp.int32),
            pltpu.VMEM((2, CH), jnp.int32),
            pltpu.VMEM((2, CH), jnp.int32),
            pltpu.VMEM((2, CH), jnp.int32),
            pltpu.VMEM((2, CH), jnp.float32),
            pltpu.VMEM((CH, D), jnp.float32),
            pltpu.VMEM((CH, D), jnp.float32),
            pltpu.VMEM_SHARED((HP, D), jnp.float32),
            pltpu.VMEM_SHARED((np_pair, D), jnp.float32),
            pltpu.SemaphoreType.DMA,
            pltpu.SemaphoreType.DMA,
            pltpu.SemaphoreType.DMA,
            pltpu.SemaphoreType.DMA,
            *[pltpu.SemaphoreType.DMA for _ in range(RING)],
        ],
    )
    def spmm(hp_hbm, sd_hbm, wf_hbm, out_hbm,
             sdr, wr, srcf, dstf, sparf, dparf, wff, rows0, rows1,
             h_sh, acc_sh, g0, g1, s0, s1, i0, i1, i2, i3):
        rows = [rows0, rows1]
        gsem = [g0, g1]
        ssem = [s0, s1]
        isem = [i0, i1, i2, i3]
        c = lax.axis_index("c")
        s = lax.axis_index("s")

        def fetch(j, r):
            pltpu.async_copy(sd_hbm.at[s, j], sdr.at[r], isem[r])
            pltpu.async_copy(wf_hbm.at[s, j], wr.at[r], isem[r])

        def wait_fetch(j, r):
            pltpu.make_async_copy(sd_hbm.at[s, j], sdr.at[r], isem[r]).wait()
            pltpu.make_async_copy(wf_hbm.at[s, j], wr.at[r], isem[r]).wait()

        def unpack(p, r):
            for v8 in range(CH // L):
                sl = pl.ds(v8 * L, L)
                sdv = sdr[r, sl]
                srcv = lax.shift_right_logical(sdv, SH)
                dstv = jnp.bitwise_and(sdv, DMASK)
                srcf[p, sl] = lax.shift_right_logical(srcv, 1)
                sparf[p, sl] = jnp.bitwise_and(srcv, 1)
                dstf[p, sl] = lax.shift_right_logical(dstv, 1)
                dparf[p, sl] = jnp.bitwise_and(dstv, 1)
                wff[p, sl] = wr[r, sl]

        def fire_gather(p):
            pltpu.async_copy(h_sh.at[srcf.at[p]], rows[p], gsem[p])

        def wait_gather(p):
            pltpu.make_async_copy(
                h_sh.at[srcf.at[p]], rows[p], gsem[p]
            ).wait()

        def fire_scatter(p):
            pltpu.async_copy(
                rows[p], acc_sh.at[dstf.at[p]], ssem[p], add=True
            )

        def wait_scatter(p):
            pltpu.make_async_copy(
                rows[p], acc_sh.at[dstf.at[p]], ssem[p]
            ).wait()

        Dh = D // 2

        def scale(p):
            rq = rows[p]

            def group(g, _):
                gs = pl.ds(g * L, L)
                wgrp = wff[p, gs]
                sgrp = sparf[p, gs]
                dgrp = dparf[p, gs]
                for i in range(L):
                    wv = _bcast(wgrp, i)
                    svf = _bcast(sgrp, i).astype(jnp.float32)
                    dvf = _bcast(dgrp, i).astype(jnp.float32)
                    fa = wv - wv * dvf
                    fb = wv * dvf
                    e = g * L + i
                    for k in range(Dh // L):
                        sla = pl.ds(k * L, L)
                        slb = pl.ds(Dh + k * L, L)
                        a = rq[e, sla]
                        bb = rq[e, slb]
                        sel = a + (bb - a) * svf
                        rq[e, sla] = sel * fa
                        rq[e, slb] = sel * fb
                return 0

            lax.fori_loop(0, CH // L, group, 0)

        pltpu.sync_copy(
            hp_hbm.at[c, pl.ds(s * SEG, SEG)], h_sh.at[pl.ds(s * SEG, SEG)]
        )

        def zrow(i, _):
            for chk in range(D // L):
                rows0[i, pl.ds(chk * L, L)] = jnp.zeros((L,), jnp.float32)
            return 0

        lax.fori_loop(0, CH, zrow, 0)
        base = s * AS
        nfull = AS // CH
        for r in range(nfull):
            pltpu.sync_copy(rows0, acc_sh.at[pl.ds(base + r * CH, CH)])
        rem = AS - nfull * CH
        if rem:
            pltpu.sync_copy(
                rows0.at[pl.ds(0, rem)],
                acc_sh.at[pl.ds(base + nfull * CH, rem)],
            )
        plsc.subcore_barrier()

        for jj in range(min(3, K)):
            fetch(jj, jj)
        wait_fetch(0, 0)
        unpack(0, 0)
        fire_gather(0)

        def outer(jo, _):
            for p4 in range(4):
                j = jo * 4 + p4
                p = p4 % 2
                q = 1 - p

                @pl.when(j + 1 < K)
                def _():
                    wait_fetch(j + 1, (p4 + 1) % RING)

                    @pl.when(j >= 1)
                    def _():
                        wait_scatter(q)

                    unpack(q, (p4 + 1) % RING)
                    fire_gather(q)

                wait_gather(p)
                scale(p)
                fire_scatter(p)

                @pl.when(j + 3 < K)
                def _():
                    fetch(j + 3, (p4 + 3) % RING)
            return 0

        lax.fori_loop(0, K // 4, outer, 0)

        wait_scatter(0)
        wait_scatter(1)
        plsc.subcore_barrier()

        pltpu.sync_copy(
            acc_sh.at[pl.ds(base, AS)],
            out_hbm.at[c, pl.ds(base, AS)],
        )

    return spmm(hpair, sd3, wf3)


def kernel(x, edge_index, edge_weight, W, b):
    n, d_in = x.shape
    d_out = W.shape[0]
    dh = d_out // 2
    e = edge_weight.shape[0]

    h = _linear(x, W.T, b.reshape(1, d_out))

    npr = -(-((n + 1) // 2) // (NS * 8)) * (NS * 8)
    hpad = jnp.concatenate(
        [h, jnp.zeros((2 * npr - n, d_out), jnp.float32)]
    )
    hpair = (
        hpad.reshape(npr, 2, 2, dh)
        .transpose(2, 0, 1, 3)
        .reshape(NC, npr, 2 * dh)
    )

    EPT = -(-e // (NS * CH * 4)) * (CH * 4)
    e_pad = NS * EPT
    pad = e_pad - e
    src = jnp.concatenate([edge_index[0], jnp.zeros((pad,), jnp.int32)])
    dst = jnp.concatenate([edge_index[1], jnp.zeros((pad,), jnp.int32)])
    w = jnp.concatenate([edge_weight, jnp.zeros((pad,), jnp.float32)])
    sd = jnp.bitwise_or(lax.shift_left(src, SH), dst)
    sd3 = sd.reshape(NS, EPT // CH, CH)
    wf3 = w.reshape(NS, EPT // CH, CH)

    partial = _spmm_sc(hpair, sd3, wf3, npr)

    p0 = partial[0].reshape(2 * npr, dh)[:n]
    p1 = partial[1].reshape(2 * npr, dh)[:n]
    return _concat_relu(p0, p1, n)
